# baseline (speedup 1.0000x reference)
import functools

import jax
import jax.numpy as jnp
from jax import lax
from jax.experimental import pallas as pl
from jax.experimental.pallas import tpu as pltpu
from jax.experimental.pallas import tpu_sc as plsc

N = 320000
D = 128
S = 10000
P = 2
NROWS = N // P
NC = 2
NS = 16
L = 16
H = D // NC
HC = H + L
CH = 128
RPT = NROWS // NS
NFULL = RPT // CH
TAIL = RPT - NFULL * CH
NBUF = 3
SP = 10240
SPT = SP // NS
FB = SPT // 4

BLK = 16000
CB = 2048

assert NFULL % NBUF == 0 and SPT % FB == 0


def _mm_body(x_ref, w_ref, b_ref, out_ref):
    h = lax.dot_general(x_ref[...], w_ref[...],
                        (((1,), (1,)), ((), ())),
                        preferred_element_type=jnp.float32)
    out_ref[...] = jnp.maximum(h + b_ref[...], 0.0)


def _make_tc_linear(part):
    blk0 = part * (NROWS // BLK)
    return pl.pallas_call(
        _mm_body,
        grid=(NROWS // BLK,),
        in_specs=[
            pl.BlockSpec((BLK, D), lambda i: (i + blk0, 0)),
            pl.BlockSpec((D, D), lambda i: (0, 0)),
            pl.BlockSpec((1, D), lambda i: (0, 0)),
        ],
        out_specs=pl.BlockSpec((BLK, D), lambda i: (i, 0)),
        out_shape=jax.ShapeDtypeStruct((NROWS, D), jnp.float32),
    )


_mesh = plsc.VectorSubcoreMesh(core_axis_name="c", subcore_axis_name="s",
                               num_cores=NC, num_subcores=NS)


def _make_sc_agg(part):
    @functools.partial(
        pl.kernel,
        out_type=(jax.ShapeDtypeStruct((SP, D), jnp.float32),
                  jax.ShapeDtypeStruct((SP, L), jnp.float32)),
        mesh=_mesh,
        scratch_types=[
            pltpu.VMEM_SHARED((SP, HC), jnp.float32),
            pltpu.VMEM((FB, HC), jnp.float32),
            pltpu.VMEM((NBUF, CH, HC), jnp.float32),
            pltpu.VMEM((NBUF, CH), jnp.int32),
            pltpu.VMEM((TAIL,), jnp.int32),
            [pltpu.SemaphoreType.DMA] * NBUF,
            [pltpu.SemaphoreType.DMA] * NBUF,
            [pltpu.SemaphoreType.DMA] * NBUF,
        ],
        compiler_params=pltpu.CompilerParams(use_tc_tiling_on_sc=False),
    )
    def _sc_agg(hp, bi, osum, ocnt, acc, zbuf, hbuf, ibuf, tibuf,
                slh, sli, ssa):
        c = lax.axis_index("c")
        s = lax.axis_index("s")
        seg0 = s * SPT
        col0 = c * H
        row0 = s * RPT
        brow0 = part * NROWS + row0

        zero = jnp.zeros((L,), jnp.float32)
        one = jnp.ones((L,), jnp.float32)

        def zero_body(i, _):
            for j in range(HC // L):
                zbuf[i, pl.ds(j * L, L)] = zero
            return 0
        lax.fori_loop(0, FB, zero_body, 0)

        def ones_body(i, _):
            for b in range(NBUF):
                hbuf[b, i, pl.ds(H, L)] = one
            return 0
        lax.fori_loop(0, CH, ones_body, 0)

        for r in range(SPT // FB):
            pltpu.sync_copy(zbuf, acc.at[pl.ds(seg0 + r * FB, FB)])
        plsc.subcore_barrier()

        def issue_loads(i, b):
            pltpu.async_copy(hp.at[pl.ds(row0 + i * CH, CH),
                                   pl.ds(col0, H)],
                             hbuf.at[b].at[:, pl.ds(0, H)], slh[b])
            pltpu.async_copy(bi.at[pl.ds(brow0 + i * CH, CH)],
                             ibuf.at[b], sli[b])

        def wait_loads(b):
            pltpu.make_async_copy(hp.at[pl.ds(row0, CH), pl.ds(col0, H)],
                                  hbuf.at[b].at[:, pl.ds(0, H)],
                                  slh[b]).wait()
            pltpu.make_async_copy(bi.at[pl.ds(brow0, CH)], ibuf.at[b],
                                  sli[b]).wait()

        for b in range(NBUF):
            issue_loads(b, b)

        def body(j, _):
            i0 = j * NBUF
            descs = []
            for b in range(NBUF):
                wait_loads(b)
                descs.append(pltpu.async_copy(hbuf.at[b],
                                              acc.at[ibuf.at[b]],
                                              ssa[b], add=True))
            for b in range(NBUF):
                descs[b].wait()
                nxt = i0 + NBUF + b

                @pl.when(nxt < NFULL)
                def _(b=b, nxt=nxt):
                    issue_loads(nxt, b)
            return 0

        lax.fori_loop(0, NFULL // NBUF, body, 0)

        rt = NFULL * CH
        pltpu.sync_copy(bi.at[pl.ds(brow0 + rt, TAIL)], tibuf)
        pltpu.sync_copy(hp.at[pl.ds(row0 + rt, TAIL), pl.ds(col0, H)],
                        hbuf.at[0].at[pl.ds(0, TAIL), pl.ds(0, H)])
        pltpu.sync_copy(hbuf.at[0].at[pl.ds(0, TAIL)], acc.at[tibuf],
                        add=True)
        plsc.subcore_barrier()

        pltpu.sync_copy(acc.at[pl.ds(seg0, SPT), pl.ds(0, H)],
                        osum.at[pl.ds(seg0, SPT), pl.ds(col0, H)])

        @pl.when(c == 0)
        def _():
            pltpu.sync_copy(acc.at[pl.ds(seg0, SPT), pl.ds(H, L)],
                            ocnt.at[pl.ds(seg0, SPT)])

    return _sc_agg


def _comb_body(*refs):
    sums = refs[:P]
    cnts = refs[P:2 * P]
    out_ref = refs[2 * P]
    total = sums[0][...]
    for p in range(1, P):
        total = total + sums[p][...]
    cn = cnts[0][...][:, :1]
    for p in range(1, P):
        cn = cn + cnts[p][...][:, :1]
    out_ref[...] = total / jnp.maximum(cn, 1.0)


def _combine(sums, cnts):
    return pl.pallas_call(
        _comb_body,
        grid=(SP // CB,),
        in_specs=[pl.BlockSpec((CB, D), lambda i: (i, 0))] * P
        + [pl.BlockSpec((CB, L), lambda i: (i, 0))] * P,
        out_specs=pl.BlockSpec((CB, D), lambda i: (i, 0)),
        out_shape=jax.ShapeDtypeStruct((SP, D), jnp.float32),
    )(*sums, *cnts)


_tc_parts = [_make_tc_linear(p) for p in range(P)]
_sc_parts = [_make_sc_agg(p) for p in range(P)]


def kernel(x, batch_index, W, b):
    bi = batch_index.astype(jnp.int32)
    b2 = b.reshape(1, D)
    sums, cnts = [], []
    for p in range(P):
        hp = _tc_parts[p](x, W, b2)
        osum, ocnt = _sc_parts[p](hp, bi)
        sums.append(osum)
        cnts.append(ocnt)
    out2 = _combine(sums, cnts)
    return out2[:S]


# speedup vs baseline: 1.0545x; 1.0545x over previous
import functools

import jax
import jax.numpy as jnp
from jax import lax
from jax.experimental import pallas as pl
from jax.experimental.pallas import tpu as pltpu
from jax.experimental.pallas import tpu_sc as plsc

N = 320000
D = 128
S = 10000
P = 2
NROWS = N // P
NC = 2
NS = 16
L = 16
H = D // NC
HC = H + L
CH = 128
RPT = NROWS // NS
NFULL = RPT // CH
TAIL = RPT - NFULL * CH
NBUF = 6
SP = 10240
SPT = SP // NS
FB = SPT // 4

BLK = 16000
CB = 2048

assert NFULL % NBUF == 0 and SPT % FB == 0


def _mm_body(x_ref, w_ref, b_ref, out_ref):
    h = lax.dot_general(x_ref[...], w_ref[...],
                        (((1,), (1,)), ((), ())),
                        preferred_element_type=jnp.float32)
    out_ref[...] = jnp.maximum(h + b_ref[...], 0.0)


def _make_tc_linear(part):
    blk0 = part * (NROWS // BLK)
    return pl.pallas_call(
        _mm_body,
        grid=(NROWS // BLK,),
        in_specs=[
            pl.BlockSpec((BLK, D), lambda i: (i + blk0, 0)),
            pl.BlockSpec((D, D), lambda i: (0, 0)),
            pl.BlockSpec((1, D), lambda i: (0, 0)),
        ],
        out_specs=pl.BlockSpec((BLK, D), lambda i: (i, 0)),
        out_shape=jax.ShapeDtypeStruct((NROWS, D), jnp.float32),
    )


_mesh = plsc.VectorSubcoreMesh(core_axis_name="c", subcore_axis_name="s",
                               num_cores=NC, num_subcores=NS)


def _make_sc_agg(part):
    @functools.partial(
        pl.kernel,
        out_type=(jax.ShapeDtypeStruct((SP, D), jnp.float32),
                  jax.ShapeDtypeStruct((SP, L), jnp.float32)),
        mesh=_mesh,
        scratch_types=[
            pltpu.VMEM_SHARED((SP, HC), jnp.float32),
            pltpu.VMEM((FB, HC), jnp.float32),
            pltpu.VMEM((NBUF, CH, HC), jnp.float32),
            pltpu.VMEM((NBUF, CH), jnp.int32),
            pltpu.VMEM((TAIL,), jnp.int32),
            [pltpu.SemaphoreType.DMA] * NBUF,
            [pltpu.SemaphoreType.DMA] * NBUF,
            [pltpu.SemaphoreType.DMA] * NBUF,
        ],
        compiler_params=pltpu.CompilerParams(use_tc_tiling_on_sc=False),
    )
    def _sc_agg(hp, bi, osum, ocnt, acc, zbuf, hbuf, ibuf, tibuf,
                slh, sli, ssa):
        c = lax.axis_index("c")
        s = lax.axis_index("s")
        seg0 = s * SPT
        col0 = c * H
        row0 = s * RPT
        brow0 = part * NROWS + row0

        zero = jnp.zeros((L,), jnp.float32)
        one = jnp.ones((L,), jnp.float32)

        def zero_body(i, _):
            for j in range(HC // L):
                zbuf[i, pl.ds(j * L, L)] = zero
            return 0
        lax.fori_loop(0, FB, zero_body, 0)

        def ones_body(i, _):
            for b in range(NBUF):
                hbuf[b, i, pl.ds(H, L)] = one
            return 0
        lax.fori_loop(0, CH, ones_body, 0)

        for r in range(SPT // FB):
            pltpu.sync_copy(zbuf, acc.at[pl.ds(seg0 + r * FB, FB)])
        plsc.subcore_barrier()

        def issue_loads(i, b):
            pltpu.async_copy(hp.at[pl.ds(row0 + i * CH, CH),
                                   pl.ds(col0, H)],
                             hbuf.at[b].at[:, pl.ds(0, H)], slh[b])
            pltpu.async_copy(bi.at[pl.ds(brow0 + i * CH, CH)],
                             ibuf.at[b], sli[b])

        def wait_loads(b):
            pltpu.make_async_copy(hp.at[pl.ds(row0, CH), pl.ds(col0, H)],
                                  hbuf.at[b].at[:, pl.ds(0, H)],
                                  slh[b]).wait()
            pltpu.make_async_copy(bi.at[pl.ds(brow0, CH)], ibuf.at[b],
                                  sli[b]).wait()

        for b in range(NBUF):
            issue_loads(b, b)

        def body(j, _):
            i0 = j * NBUF
            descs = []
            for b in range(NBUF):
                wait_loads(b)
                descs.append(pltpu.async_copy(hbuf.at[b],
                                              acc.at[ibuf.at[b]],
                                              ssa[b], add=True))
            for b in range(NBUF):
                descs[b].wait()
                nxt = i0 + NBUF + b

                @pl.when(nxt < NFULL)
                def _(b=b, nxt=nxt):
                    issue_loads(nxt, b)
            return 0

        lax.fori_loop(0, NFULL // NBUF, body, 0)

        rt = NFULL * CH
        pltpu.sync_copy(bi.at[pl.ds(brow0 + rt, TAIL)], tibuf)
        pltpu.sync_copy(hp.at[pl.ds(row0 + rt, TAIL), pl.ds(col0, H)],
                        hbuf.at[0].at[pl.ds(0, TAIL), pl.ds(0, H)])
        pltpu.sync_copy(hbuf.at[0].at[pl.ds(0, TAIL)], acc.at[tibuf],
                        add=True)
        plsc.subcore_barrier()

        pltpu.sync_copy(acc.at[pl.ds(seg0, SPT), pl.ds(0, H)],
                        osum.at[pl.ds(seg0, SPT), pl.ds(col0, H)])

        @pl.when(c == 0)
        def _():
            pltpu.sync_copy(acc.at[pl.ds(seg0, SPT), pl.ds(H, L)],
                            ocnt.at[pl.ds(seg0, SPT)])

    return _sc_agg


def _comb_body(*refs):
    sums = refs[:P]
    cnts = refs[P:2 * P]
    out_ref = refs[2 * P]
    total = sums[0][...]
    for p in range(1, P):
        total = total + sums[p][...]
    cn = cnts[0][...][:, :1]
    for p in range(1, P):
        cn = cn + cnts[p][...][:, :1]
    out_ref[...] = total / jnp.maximum(cn, 1.0)


def _combine(sums, cnts):
    return pl.pallas_call(
        _comb_body,
        grid=(SP // CB,),
        in_specs=[pl.BlockSpec((CB, D), lambda i: (i, 0))] * P
        + [pl.BlockSpec((CB, L), lambda i: (i, 0))] * P,
        out_specs=pl.BlockSpec((CB, D), lambda i: (i, 0)),
        out_shape=jax.ShapeDtypeStruct((SP, D), jnp.float32),
    )(*sums, *cnts)


_tc_parts = [_make_tc_linear(p) for p in range(P)]
_sc_parts = [_make_sc_agg(p) for p in range(P)]


def kernel(x, batch_index, W, b):
    bi = batch_index.astype(jnp.int32)
    b2 = b.reshape(1, D)
    sums, cnts = [], []
    for p in range(P):
        hp = _tc_parts[p](x, W, b2)
        osum, ocnt = _sc_parts[p](hp, bi)
        sums.append(osum)
        cnts.append(ocnt)
    out2 = _combine(sums, cnts)
    return out2[:S]


# speedup vs baseline: 1.1197x; 1.0618x over previous
import functools

import jax
import jax.numpy as jnp
from jax import lax
from jax.experimental import pallas as pl
from jax.experimental.pallas import tpu as pltpu
from jax.experimental.pallas import tpu_sc as plsc

N = 320000
D = 128
S = 10000
P = 2
NROWS = N // P
NC = 2
NS = 16
L = 16
H = D // NC
CH = 128
RPT = NROWS // NS
NFULL = RPT // CH
TAIL = RPT - NFULL * CH
NBUF = 6
SP = 10240
SPT = SP // NS
FB = SPT // 4

BLK = 16000

assert NFULL % NBUF == 0 and SPT % FB == 0


def _mm_body(x_ref, w_ref, b_ref, out_ref):
    h = lax.dot_general(x_ref[...], w_ref[...],
                        (((1,), (1,)), ((), ())),
                        preferred_element_type=jnp.float32)
    out_ref[...] = jnp.maximum(h + b_ref[...], 0.0)


def _make_tc_linear(part):
    blk0 = part * (NROWS // BLK)
    return pl.pallas_call(
        _mm_body,
        grid=(NROWS // BLK,),
        in_specs=[
            pl.BlockSpec((BLK, D), lambda i: (i + blk0, 0)),
            pl.BlockSpec((D, D), lambda i: (0, 0)),
            pl.BlockSpec((1, D), lambda i: (0, 0)),
        ],
        out_specs=pl.BlockSpec((BLK, D), lambda i: (i, 0)),
        out_shape=jax.ShapeDtypeStruct((NROWS, D), jnp.float32),
    )


_mesh = plsc.VectorSubcoreMesh(core_axis_name="c", subcore_axis_name="s",
                               num_cores=NC, num_subcores=NS)

_SC_SCRATCH = [
    pltpu.VMEM_SHARED((SP, H), jnp.float32),
    pltpu.VMEM_SHARED((SP, L), jnp.float32),
    pltpu.VMEM((FB, H), jnp.float32),
    pltpu.VMEM((FB, L), jnp.float32),
    pltpu.VMEM((NBUF, CH, H), jnp.float32),
    pltpu.VMEM((NBUF, CH), jnp.int32),
    pltpu.VMEM((TAIL,), jnp.int32),
    pltpu.VMEM((CH, L), jnp.float32),
    [pltpu.SemaphoreType.DMA] * NBUF,
    [pltpu.SemaphoreType.DMA] * NBUF,
    [pltpu.SemaphoreType.DMA] * NBUF,
    [pltpu.SemaphoreType.DMA] * NBUF,
]


def _sc_main(part, hp, bi, acc, cnt, zbuf, czbuf, hbuf, ibuf, tibuf, ones,
             slh, sli, ssa, ssc, c, s):
    seg0 = s * SPT
    col0 = c * H
    row0 = s * RPT
    brow0 = part * NROWS + row0

    zero = jnp.zeros((L,), jnp.float32)
    one = jnp.ones((L,), jnp.float32)

    def zero_body(i, _):
        for j in range(H // L):
            zbuf[i, pl.ds(j * L, L)] = zero
        czbuf[i, :] = zero
        return 0
    lax.fori_loop(0, FB, zero_body, 0)

    def ones_body(i, _):
        ones[i, :] = one
        return 0
    lax.fori_loop(0, CH, ones_body, 0)

    for r in range(SPT // FB):
        pltpu.sync_copy(zbuf, acc.at[pl.ds(seg0 + r * FB, FB)])
        pltpu.sync_copy(czbuf, cnt.at[pl.ds(seg0 + r * FB, FB)])
    plsc.subcore_barrier()

    def issue_loads(i, b):
        pltpu.async_copy(hp.at[pl.ds(row0 + i * CH, CH),
                               pl.ds(col0, H)], hbuf.at[b], slh[b])
        pltpu.async_copy(bi.at[pl.ds(brow0 + i * CH, CH)],
                         ibuf.at[b], sli[b])

    def wait_loads(b):
        pltpu.make_async_copy(hp.at[pl.ds(row0, CH), pl.ds(col0, H)],
                              hbuf.at[b], slh[b]).wait()
        pltpu.make_async_copy(bi.at[pl.ds(brow0, CH)], ibuf.at[b],
                              sli[b]).wait()

    def issue_scatters(b):
        sa = pltpu.async_copy(hbuf.at[b], acc.at[ibuf.at[b]],
                              ssa[b], add=True)
        sc = pltpu.async_copy(ones, cnt.at[ibuf.at[b]], ssc[b], add=True)
        return sa, sc

    for b in range(NBUF):
        issue_loads(b, b)

    def body(j, _):
        i0 = j * NBUF
        descs = []
        for b in range(NBUF):
            wait_loads(b)
            descs.append(issue_scatters(b))
        for b in range(NBUF):
            descs[b][0].wait()
            descs[b][1].wait()
            nxt = i0 + NBUF + b

            @pl.when(nxt < NFULL)
            def _(b=b, nxt=nxt):
                issue_loads(nxt, b)
        return 0

    lax.fori_loop(0, NFULL // NBUF, body, 0)

    rt = NFULL * CH
    pltpu.sync_copy(bi.at[pl.ds(brow0 + rt, TAIL)], tibuf)
    pltpu.sync_copy(hp.at[pl.ds(row0 + rt, TAIL), pl.ds(col0, H)],
                    hbuf.at[0].at[pl.ds(0, TAIL)])
    pltpu.sync_copy(hbuf.at[0].at[pl.ds(0, TAIL)], acc.at[tibuf], add=True)
    pltpu.sync_copy(ones.at[pl.ds(0, TAIL)], cnt.at[tibuf], add=True)
    plsc.subcore_barrier()


def _make_sc_partial(part):
    @functools.partial(
        pl.kernel,
        out_type=(jax.ShapeDtypeStruct((SP, D), jnp.float32),
                  jax.ShapeDtypeStruct((SP, L), jnp.float32)),
        mesh=_mesh,
        scratch_types=_SC_SCRATCH,
        compiler_params=pltpu.CompilerParams(use_tc_tiling_on_sc=False),
    )
    def _sc_agg(hp, bi, osum, ocnt, acc, cnt, zbuf, czbuf,
                hbuf, ibuf, tibuf, ones, slh, sli, ssa, ssc):
        c = lax.axis_index("c")
        s = lax.axis_index("s")
        _sc_main(part, hp, bi, acc, cnt, zbuf, czbuf, hbuf, ibuf, tibuf,
                 ones, slh, sli, ssa, ssc, c, s)
        seg0 = s * SPT
        col0 = c * H
        pltpu.sync_copy(acc.at[pl.ds(seg0, SPT)],
                        osum.at[pl.ds(seg0, SPT), pl.ds(col0, H)])

        @pl.when(c == 0)
        def _():
            pltpu.sync_copy(cnt.at[pl.ds(seg0, SPT)],
                            ocnt.at[pl.ds(seg0, SPT)])

    return _sc_agg


CB = 2048


def _comb_body(*refs):
    sums = refs[:P]
    cnts = refs[P:2 * P]
    out_ref = refs[2 * P]
    total = sums[0][...]
    for p in range(1, P):
        total = total + sums[p][...]
    cn = cnts[0][...][:, :1]
    for p in range(1, P):
        cn = cn + cnts[p][...][:, :1]
    out_ref[...] = total / jnp.maximum(cn, 1.0)


def _combine(sums, cnts):
    return pl.pallas_call(
        _comb_body,
        grid=(SP // CB,),
        in_specs=[pl.BlockSpec((CB, D), lambda i: (i, 0))] * P
        + [pl.BlockSpec((CB, L), lambda i: (i, 0))] * P,
        out_specs=pl.BlockSpec((CB, D), lambda i: (i, 0)),
        out_shape=jax.ShapeDtypeStruct((SP, D), jnp.float32),
    )(*sums, *cnts)


_tc_parts = [_make_tc_linear(p) for p in range(P)]
_sc_parts = [_make_sc_partial(p) for p in range(P)]


def kernel(x, batch_index, W, b):
    bi = batch_index.astype(jnp.int32)
    b2 = b.reshape(1, D)
    sums, cnts = [], []
    for p in range(P):
        hp = _tc_parts[p](x, W, b2)
        osum, ocnt = _sc_parts[p](hp, bi)
        sums.append(osum)
        cnts.append(ocnt)
    out2 = _combine(sums, cnts)
    return out2[:S]


# speedup vs baseline: 1.1286x; 1.0079x over previous
import functools

import jax
import jax.numpy as jnp
from jax import lax
from jax.experimental import pallas as pl
from jax.experimental.pallas import tpu as pltpu
from jax.experimental.pallas import tpu_sc as plsc

N = 320000
D = 128
S = 10000
P = 2
NROWS = N // P
NC = 2
NS = 16
L = 16
H = D // NC
CH = 128
RPT = NROWS // NS
NFULL = RPT // CH
TAIL = RPT - NFULL * CH
NBUF = 6
SP = 10240
SPT = SP // NS
FB = SPT // 4

BLK = 20000

assert NFULL % NBUF == 0 and SPT % FB == 0


def _mm_body(x_ref, w_ref, b_ref, out_ref):
    h = lax.dot_general(x_ref[...], w_ref[...],
                        (((1,), (1,)), ((), ())),
                        preferred_element_type=jnp.float32)
    out_ref[...] = jnp.maximum(h + b_ref[...], 0.0)


def _make_tc_linear(part):
    blk0 = part * (NROWS // BLK)
    return pl.pallas_call(
        _mm_body,
        grid=(NROWS // BLK,),
        in_specs=[
            pl.BlockSpec((BLK, D), lambda i: (i + blk0, 0)),
            pl.BlockSpec((D, D), lambda i: (0, 0)),
            pl.BlockSpec((1, D), lambda i: (0, 0)),
        ],
        out_specs=pl.BlockSpec((BLK, D), lambda i: (i, 0)),
        out_shape=jax.ShapeDtypeStruct((NROWS, D), jnp.float32),
    )


_mesh = plsc.VectorSubcoreMesh(core_axis_name="c", subcore_axis_name="s",
                               num_cores=NC, num_subcores=NS)

_SC_SCRATCH = [
    pltpu.VMEM_SHARED((SP, H), jnp.float32),
    pltpu.VMEM_SHARED((SP, L), jnp.float32),
    pltpu.VMEM((FB, H), jnp.float32),
    pltpu.VMEM((FB, L), jnp.float32),
    pltpu.VMEM((NBUF, CH, H), jnp.float32),
    pltpu.VMEM((NBUF, CH), jnp.int32),
    pltpu.VMEM((TAIL,), jnp.int32),
    pltpu.VMEM((CH, L), jnp.float32),
    [pltpu.SemaphoreType.DMA] * NBUF,
    [pltpu.SemaphoreType.DMA] * NBUF,
    [pltpu.SemaphoreType.DMA] * NBUF,
    [pltpu.SemaphoreType.DMA] * NBUF,
]


def _sc_main(part, hp, bi, acc, cnt, zbuf, czbuf, hbuf, ibuf, tibuf, ones,
             slh, sli, ssa, ssc, c, s):
    seg0 = s * SPT
    col0 = c * H
    row0 = s * RPT
    brow0 = part * NROWS + row0

    zero = jnp.zeros((L,), jnp.float32)
    one = jnp.ones((L,), jnp.float32)

    def zero_body(i, _):
        for j in range(H // L):
            zbuf[i, pl.ds(j * L, L)] = zero
        czbuf[i, :] = zero
        return 0
    lax.fori_loop(0, FB, zero_body, 0)

    def ones_body(i, _):
        ones[i, :] = one
        return 0
    lax.fori_loop(0, CH, ones_body, 0)

    for r in range(SPT // FB):
        pltpu.sync_copy(zbuf, acc.at[pl.ds(seg0 + r * FB, FB)])
        pltpu.sync_copy(czbuf, cnt.at[pl.ds(seg0 + r * FB, FB)])
    plsc.subcore_barrier()

    def issue_loads(i, b):
        pltpu.async_copy(hp.at[pl.ds(row0 + i * CH, CH),
                               pl.ds(col0, H)], hbuf.at[b], slh[b])
        pltpu.async_copy(bi.at[pl.ds(brow0 + i * CH, CH)],
                         ibuf.at[b], sli[b])

    def wait_loads(b):
        pltpu.make_async_copy(hp.at[pl.ds(row0, CH), pl.ds(col0, H)],
                              hbuf.at[b], slh[b]).wait()
        pltpu.make_async_copy(bi.at[pl.ds(brow0, CH)], ibuf.at[b],
                              sli[b]).wait()

    def issue_scatters(b):
        sa = pltpu.async_copy(hbuf.at[b], acc.at[ibuf.at[b]],
                              ssa[b], add=True)
        sc = pltpu.async_copy(ones, cnt.at[ibuf.at[b]], ssc[b], add=True)
        return sa, sc

    for b in range(NBUF):
        issue_loads(b, b)

    def body(j, _):
        i0 = j * NBUF
        descs = []
        for b in range(NBUF):
            wait_loads(b)
            descs.append(issue_scatters(b))
        for b in range(NBUF):
            descs[b][0].wait()
            descs[b][1].wait()
            nxt = i0 + NBUF + b

            @pl.when(nxt < NFULL)
            def _(b=b, nxt=nxt):
                issue_loads(nxt, b)
        return 0

    lax.fori_loop(0, NFULL // NBUF, body, 0)

    rt = NFULL * CH
    pltpu.sync_copy(bi.at[pl.ds(brow0 + rt, TAIL)], tibuf)
    pltpu.sync_copy(hp.at[pl.ds(row0 + rt, TAIL), pl.ds(col0, H)],
                    hbuf.at[0].at[pl.ds(0, TAIL)])
    pltpu.sync_copy(hbuf.at[0].at[pl.ds(0, TAIL)], acc.at[tibuf], add=True)
    pltpu.sync_copy(ones.at[pl.ds(0, TAIL)], cnt.at[tibuf], add=True)
    plsc.subcore_barrier()


def _make_sc_partial(part):
    @functools.partial(
        pl.kernel,
        out_type=(jax.ShapeDtypeStruct((SP, D), jnp.float32),
                  jax.ShapeDtypeStruct((SP, L), jnp.float32)),
        mesh=_mesh,
        scratch_types=_SC_SCRATCH,
        compiler_params=pltpu.CompilerParams(use_tc_tiling_on_sc=False),
    )
    def _sc_agg(hp, bi, osum, ocnt, acc, cnt, zbuf, czbuf,
                hbuf, ibuf, tibuf, ones, slh, sli, ssa, ssc):
        c = lax.axis_index("c")
        s = lax.axis_index("s")
        _sc_main(part, hp, bi, acc, cnt, zbuf, czbuf, hbuf, ibuf, tibuf,
                 ones, slh, sli, ssa, ssc, c, s)
        seg0 = s * SPT
        col0 = c * H
        pltpu.sync_copy(acc.at[pl.ds(seg0, SPT)],
                        osum.at[pl.ds(seg0, SPT), pl.ds(col0, H)])

        @pl.when(c == 0)
        def _():
            pltpu.sync_copy(cnt.at[pl.ds(seg0, SPT)],
                            ocnt.at[pl.ds(seg0, SPT)])

    return _sc_agg


CB = 5120


def _comb_body(*refs):
    sums = refs[:P]
    cnts = refs[P:2 * P]
    out_ref = refs[2 * P]
    total = sums[0][...]
    for p in range(1, P):
        total = total + sums[p][...]
    cn = cnts[0][...][:, :1]
    for p in range(1, P):
        cn = cn + cnts[p][...][:, :1]
    out_ref[...] = total / jnp.maximum(cn, 1.0)


def _combine(sums, cnts):
    return pl.pallas_call(
        _comb_body,
        grid=(SP // CB,),
        in_specs=[pl.BlockSpec((CB, D), lambda i: (i, 0))] * P
        + [pl.BlockSpec((CB, L), lambda i: (i, 0))] * P,
        out_specs=pl.BlockSpec((CB, D), lambda i: (i, 0)),
        out_shape=jax.ShapeDtypeStruct((SP, D), jnp.float32),
    )(*sums, *cnts)


_tc_parts = [_make_tc_linear(p) for p in range(P)]
_sc_parts = [_make_sc_partial(p) for p in range(P)]


def kernel(x, batch_index, W, b):
    bi = batch_index.astype(jnp.int32)
    b2 = b.reshape(1, D)
    sums, cnts = [], []
    for p in range(P):
        hp = _tc_parts[p](x, W, b2)
        osum, ocnt = _sc_parts[p](hp, bi)
        sums.append(osum)
        cnts.append(ocnt)
    out2 = _combine(sums, cnts)
    return out2[:S]
